# baseline (speedup 1.0000x reference)
import functools

import jax
import jax.numpy as jnp
from jax import lax
from jax.experimental import pallas as pl
from jax.experimental.pallas import tpu as pltpu
from jax.experimental.pallas import tpu_sc as plsc

DIM = 32
SLOT = 128
ROWS_PER_SLOT = SLOT // DIM
CHUNK = 128


def _make_sc_gather(n_rows: int):
    info = plsc.get_sparse_core_info()
    nc, ns = info.num_cores, info.num_subcores
    nw = nc * ns
    n_chunks = n_rows // CHUNK
    chunks_per_w = n_chunks // nw
    assert chunks_per_w * nw == n_chunks

    mesh = plsc.VectorSubcoreMesh(core_axis_name="c", subcore_axis_name="s")

    @functools.partial(
        pl.kernel,
        mesh=mesh,
        out_type=jax.ShapeDtypeStruct((n_chunks, CHUNK, SLOT), jnp.float32),
        scratch_types=[
            pltpu.VMEM((chunks_per_w, CHUNK), jnp.int32),
            pltpu.VMEM((CHUNK, SLOT), jnp.float32),
            pltpu.VMEM((CHUNK, SLOT), jnp.float32),
            pltpu.SemaphoreType.DMA,
            pltpu.SemaphoreType.DMA,
        ],
    )
    def gather_kernel(table_hbm, idx_hbm, out_hbm, idx_v, buf0, buf1,
                      sem0, sem1):
        wid = lax.axis_index("s") * nc + lax.axis_index("c")
        base = wid * chunks_per_w
        pltpu.sync_copy(idx_hbm.at[pl.ds(base, chunks_per_w)], idx_v)
        bufs = (buf0, buf1)
        sems = (sem0, sem1)
        copies = [None, None]
        copies[0] = pltpu.async_copy(table_hbm.at[idx_v.at[0]], buf0, sem0)
        for j in range(chunks_per_w):
            nj = j + 1
            if nj < chunks_per_w:
                copies[nj % 2] = pltpu.async_copy(
                    table_hbm.at[idx_v.at[nj]], bufs[nj % 2], sems[nj % 2]
                )
            copies[j % 2].wait()
            pltpu.sync_copy(bufs[j % 2], out_hbm.at[base + j])

    return gather_kernel


def _select32(slots, mask):
    return (
        slots[:, 0:32] * mask[:, 0:1]
        + slots[:, 32:64] * mask[:, 1:2]
        + slots[:, 64:96] * mask[:, 2:3]
        + slots[:, 96:128] * mask[:, 3:4]
    )


def _mlp_body(us_ref, is_ref, f_ref, mu_ref, mi_ref, w1u_ref, w1i_ref,
              w1f_ref, b1_ref, w2_ref, b2_ref, o_ref):
    ue = _select32(us_ref[...], mu_ref[...])
    ie = _select32(is_ref[...], mi_ref[...])
    h = (
        jnp.dot(ue, w1u_ref[...], preferred_element_type=jnp.float32)
        + jnp.dot(ie, w1i_ref[...], preferred_element_type=jnp.float32)
        + jnp.dot(f_ref[...], w1f_ref[...], preferred_element_type=jnp.float32)
        + b1_ref[...]
    )
    h = jnp.maximum(h, 0.0)
    o_ref[...] = (
        jnp.dot(h, w2_ref[...], preferred_element_type=jnp.float32)
        + b2_ref[...]
    )


def kernel(userID, itemID, features, user_emb, item_emb, W1, b1, W2, b2):
    del item_emb
    batch = userID.shape[0]
    n_rows = 2 * batch
    num_users = user_emb.shape[0]

    idx = jnp.concatenate([userID, itemID]).astype(jnp.int32)
    slot_idx = (idx // ROWS_PER_SLOT).reshape(n_rows // CHUNK, CHUNK)
    table128 = user_emb.reshape(num_users // ROWS_PER_SLOT, SLOT)

    gathered = _make_sc_gather(n_rows)(table128, slot_idx)
    g = gathered.reshape(n_rows, SLOT)

    sub = idx % ROWS_PER_SLOT
    onehot = (sub[:, None] == jnp.arange(ROWS_PER_SLOT)[None, :]).astype(
        jnp.float32
    )
    mask_u = onehot[:batch]
    mask_i = onehot[batch:]

    blk = 2048
    nblk = batch // blk
    feat_dim = features.shape[1]
    hid = W1.shape[1]

    w1u = W1[:DIM]
    w1i = W1[DIM:2 * DIM]
    w1f = W1[2 * DIM:]
    b1r = b1.reshape(1, hid)
    b2r = b2.reshape(1, 1)

    out = pl.pallas_call(
        _mlp_body,
        grid=(nblk,),
        in_specs=[
            pl.BlockSpec((blk, SLOT), lambda i: (i, 0)),
            pl.BlockSpec((blk, SLOT), lambda i: (i + nblk, 0)),
            pl.BlockSpec((blk, feat_dim), lambda i: (i, 0)),
            pl.BlockSpec((blk, ROWS_PER_SLOT), lambda i: (i, 0)),
            pl.BlockSpec((blk, ROWS_PER_SLOT), lambda i: (i, 0)),
            pl.BlockSpec((DIM, hid), lambda i: (0, 0)),
            pl.BlockSpec((DIM, hid), lambda i: (0, 0)),
            pl.BlockSpec((feat_dim, hid), lambda i: (0, 0)),
            pl.BlockSpec((1, hid), lambda i: (0, 0)),
            pl.BlockSpec((hid, 1), lambda i: (0, 0)),
            pl.BlockSpec((1, 1), lambda i: (0, 0)),
        ],
        out_specs=pl.BlockSpec((blk, 1), lambda i: (i, 0)),
        out_shape=jax.ShapeDtypeStruct((batch, 1), jnp.float32),
    )(g, g, features, mask_u, mask_i, w1u, w1i, w1f, b1r, W2, b2r)

    return out


# speedup vs baseline: 1.3960x; 1.3960x over previous
import functools

import jax
import jax.numpy as jnp
from jax import lax
from jax.experimental import pallas as pl
from jax.experimental.pallas import tpu as pltpu
from jax.experimental.pallas import tpu_sc as plsc

DIM = 32
PW = 128
CHUNK = 128
_DN0 = (((0,), (0,)), ((), ()))


def _proj_body(x_ref, wu_ref, wi_ref, o_ref):
    x = x_ref[...]
    pu = lax.dot_general(x, wu_ref[...], _DN0,
                         preferred_element_type=jnp.float32)
    pi = lax.dot_general(x, wi_ref[...], _DN0,
                         preferred_element_type=jnp.float32)
    o_ref[...] = jnp.concatenate([pu, pi], axis=1)


def _make_sc_gather(n_rows: int, n_table_rows: int):
    info = plsc.get_sparse_core_info()
    nc, ns = info.num_cores, info.num_subcores
    nw = nc * ns
    n_chunks = n_rows // CHUNK
    chunks_per_w = n_chunks // nw
    assert chunks_per_w * nw == n_chunks

    mesh = plsc.VectorSubcoreMesh(core_axis_name="c", subcore_axis_name="s")

    @functools.partial(
        pl.kernel,
        mesh=mesh,
        out_type=jax.ShapeDtypeStruct((n_chunks, CHUNK, PW), jnp.float32),
        scratch_types=[
            pltpu.VMEM((chunks_per_w, CHUNK), jnp.int32),
            pltpu.VMEM((CHUNK, PW), jnp.float32),
            pltpu.VMEM((CHUNK, PW), jnp.float32),
            pltpu.SemaphoreType.DMA,
            pltpu.SemaphoreType.DMA,
        ],
    )
    def gather_kernel(table_hbm, idx_hbm, out_hbm, idx_v, buf0, buf1,
                      sem0, sem1):
        wid = lax.axis_index("s") * nc + lax.axis_index("c")
        base = wid * chunks_per_w
        pltpu.sync_copy(idx_hbm.at[pl.ds(base, chunks_per_w)], idx_v)
        bufs = (buf0, buf1)
        sems = (sem0, sem1)
        copies = [None, None]
        copies[0] = pltpu.async_copy(table_hbm.at[idx_v.at[0]], buf0, sem0)
        for j in range(chunks_per_w):
            nj = j + 1
            if nj < chunks_per_w:
                copies[nj % 2] = pltpu.async_copy(
                    table_hbm.at[idx_v.at[nj]], bufs[nj % 2], sems[nj % 2]
                )
            copies[j % 2].wait()
            pltpu.sync_copy(bufs[j % 2], out_hbm.at[base + j])

    return gather_kernel


def _mlp_body(gu_ref, gi_ref, ft_ref, w1f_ref, b1_ref, w2_ref, b2_ref,
              o_ref):
    hf = lax.dot_general(ft_ref[...], w1f_ref[...], _DN0,
                         preferred_element_type=jnp.float32)
    hid = hf.shape[1]
    h = gu_ref[:, :hid] + gi_ref[:, hid:2 * hid] + hf + b1_ref[...]
    h = jnp.maximum(h, 0.0)
    o_ref[...] = (
        jnp.dot(h, w2_ref[...], preferred_element_type=jnp.float32)
        + b2_ref[...]
    )


def kernel(userID, itemID, features, user_emb, item_emb, W1, b1, W2, b2):
    del item_emb
    batch = userID.shape[0]
    n_rows = 2 * batch
    num_users = user_emb.shape[0]
    hid = W1.shape[1]

    tabT = user_emb.T
    w1u = W1[:DIM]
    w1i = W1[DIM:2 * DIM]

    blkc = 4096
    gridp = pl.cdiv(num_users, blkc)
    P = pl.pallas_call(
        _proj_body,
        grid=(gridp,),
        in_specs=[
            pl.BlockSpec((DIM, blkc), lambda i: (0, i)),
            pl.BlockSpec((DIM, hid), lambda i: (0, 0)),
            pl.BlockSpec((DIM, hid), lambda i: (0, 0)),
        ],
        out_specs=pl.BlockSpec((blkc, PW), lambda i: (i, 0)),
        out_shape=jax.ShapeDtypeStruct((num_users, PW), jnp.float32),
    )(tabT, w1u, w1i)

    idx = jnp.concatenate([userID, itemID]).astype(jnp.int32)
    idx2d = idx.reshape(n_rows // CHUNK, CHUNK)
    gathered = _make_sc_gather(n_rows, num_users)(P, idx2d)
    g = gathered.reshape(n_rows, PW)

    fT = features.T
    feat_dim = fT.shape[0]
    w1f = W1[2 * DIM:]
    b1r = b1.reshape(1, hid)
    b2r = b2.reshape(1, 1)

    blk = 2048
    nblk = batch // blk

    out = pl.pallas_call(
        _mlp_body,
        grid=(nblk,),
        in_specs=[
            pl.BlockSpec((blk, PW), lambda i: (i, 0)),
            pl.BlockSpec((blk, PW), lambda i: (i + nblk, 0)),
            pl.BlockSpec((feat_dim, blk), lambda i: (0, i)),
            pl.BlockSpec((feat_dim, hid), lambda i: (0, 0)),
            pl.BlockSpec((1, hid), lambda i: (0, 0)),
            pl.BlockSpec((hid, 1), lambda i: (0, 0)),
            pl.BlockSpec((1, 1), lambda i: (0, 0)),
        ],
        out_specs=pl.BlockSpec((blk, 1), lambda i: (i, 0)),
        out_shape=jax.ShapeDtypeStruct((batch, 1), jnp.float32),
    )(g, g, fT, w1f, b1r, W2, b2r)

    return out


# speedup vs baseline: 2.1054x; 1.5082x over previous
import functools

import jax
import jax.numpy as jnp
from jax import lax
from jax.experimental import pallas as pl
from jax.experimental.pallas import tpu as pltpu
from jax.experimental.pallas import tpu_sc as plsc

DIM = 32
PW = 128
CHUNK = 128
_DN0 = (((0,), (0,)), ((), ()))


def _proj_body(x_ref, w_ref, o_ref):
    x = x_ref[...].astype(jnp.bfloat16)
    w = w_ref[...].astype(jnp.bfloat16)
    o_ref[...] = lax.dot_general(x, w, _DN0,
                                 preferred_element_type=jnp.float32)


def _make_sc_gather(n_rows: int, n_table_rows: int):
    info = plsc.get_sparse_core_info()
    nc, ns = info.num_cores, info.num_subcores
    nw = nc * ns
    n_chunks = n_rows // CHUNK
    chunks_per_w = n_chunks // nw
    assert chunks_per_w * nw == n_chunks

    mesh = plsc.VectorSubcoreMesh(core_axis_name="c", subcore_axis_name="s")

    @functools.partial(
        pl.kernel,
        mesh=mesh,
        out_type=jax.ShapeDtypeStruct((n_chunks, CHUNK, PW), jnp.float32),
        scratch_types=[
            pltpu.VMEM((chunks_per_w, CHUNK), jnp.int32),
            pltpu.VMEM((CHUNK, PW), jnp.float32),
            pltpu.VMEM((CHUNK, PW), jnp.float32),
            pltpu.SemaphoreType.DMA,
            pltpu.SemaphoreType.DMA,
        ],
    )
    def gather_kernel(table_hbm, idx_hbm, out_hbm, idx_v, buf0, buf1,
                      sem0, sem1):
        wid = lax.axis_index("s") * nc + lax.axis_index("c")
        base = wid * chunks_per_w
        pltpu.sync_copy(idx_hbm.at[pl.ds(base, chunks_per_w)], idx_v)
        bufs = (buf0, buf1)
        sems = (sem0, sem1)
        copies = [None, None]
        copies[0] = pltpu.async_copy(table_hbm.at[idx_v.at[0]], buf0, sem0)
        for j in range(chunks_per_w):
            nj = j + 1
            if nj < chunks_per_w:
                copies[nj % 2] = pltpu.async_copy(
                    table_hbm.at[idx_v.at[nj]], bufs[nj % 2], sems[nj % 2]
                )
            copies[j % 2].wait()
            pltpu.sync_copy(bufs[j % 2], out_hbm.at[base + j])

    return gather_kernel


def _mlp_body(gu_ref, gi_ref, ft_ref, w1f_ref, b1_ref, w2_ref, b2_ref,
              o_ref):
    hf = lax.dot_general(ft_ref[...], w1f_ref[...], _DN0,
                         preferred_element_type=jnp.float32)
    hid = hf.shape[1]
    h = gu_ref[:, :hid] + gi_ref[:, hid:2 * hid] + hf + b1_ref[...]
    h = jnp.maximum(h, 0.0)
    o_ref[...] = (
        jnp.dot(h, w2_ref[...], preferred_element_type=jnp.float32)
        + b2_ref[...]
    )


def kernel(userID, itemID, features, user_emb, item_emb, W1, b1, W2, b2):
    del item_emb
    batch = userID.shape[0]
    n_rows = 2 * batch
    num_users = user_emb.shape[0]
    hid = W1.shape[1]

    tabT = user_emb.T
    w_ui = W1[:2 * DIM].reshape(2, DIM, hid).transpose(1, 0, 2).reshape(
        DIM, 2 * hid)

    blkc = 8192
    gridp = pl.cdiv(num_users, blkc)
    P = pl.pallas_call(
        _proj_body,
        grid=(gridp,),
        in_specs=[
            pl.BlockSpec((DIM, blkc), lambda i: (0, i)),
            pl.BlockSpec((DIM, PW), lambda i: (0, 0)),
        ],
        out_specs=pl.BlockSpec((blkc, PW), lambda i: (i, 0)),
        out_shape=jax.ShapeDtypeStruct((num_users, PW), jnp.float32),
    )(tabT, w_ui)

    idx = jnp.concatenate([userID, itemID]).astype(jnp.int32)
    idx2d = idx.reshape(n_rows // CHUNK, CHUNK)
    gathered = _make_sc_gather(n_rows, num_users)(P, idx2d)
    g = gathered.reshape(n_rows, PW)

    fT = features.T
    feat_dim = fT.shape[0]
    w1f = W1[2 * DIM:]
    b1r = b1.reshape(1, hid)
    b2r = b2.reshape(1, 1)

    blk = 2048
    nblk = batch // blk

    out = pl.pallas_call(
        _mlp_body,
        grid=(nblk,),
        in_specs=[
            pl.BlockSpec((blk, PW), lambda i: (i, 0)),
            pl.BlockSpec((blk, PW), lambda i: (i + nblk, 0)),
            pl.BlockSpec((feat_dim, blk), lambda i: (0, i)),
            pl.BlockSpec((feat_dim, hid), lambda i: (0, 0)),
            pl.BlockSpec((1, hid), lambda i: (0, 0)),
            pl.BlockSpec((hid, 1), lambda i: (0, 0)),
            pl.BlockSpec((1, 1), lambda i: (0, 0)),
        ],
        out_specs=pl.BlockSpec((blk, 1), lambda i: (i, 0)),
        out_shape=jax.ShapeDtypeStruct((batch, 1), jnp.float32),
    )(g, g, fT, w1f, b1r, W2, b2r)

    return out
